# baseline (speedup 1.0000x reference)
import functools

import jax
import jax.numpy as jnp
from jax.experimental import pallas as pl
from jax.experimental.pallas import tpu as pltpu

EPS_TOPO = 0.7
BB = 8


def _fused_kernel(ppl_ref, vh_ref, ans_ref, w_dh_ref, w_vh_ref, b_vh_ref,
                  b_dh_ref, w_topo_ref, wg1_ref, bg1_ref, wg2_ref,
                  bg2_ref, wg3_ref, bg3_ref, out_ref, vb_s, w_ans_s,
                  *, n_nodes, k_heads, d_ppl, d_vh, d_ans):
    f32 = jnp.float32
    n, kh = n_nodes, k_heads
    r = BB * n
    h = w_dh_ref.shape[1]

    @pl.when(pl.program_id(0) == 0)
    def _init():
        vw = jnp.concatenate(
            [w_vh_ref[...], b_vh_ref[...].reshape(1, d_vh)], axis=0)
        vb_s[...] = jnp.dot(vw, w_dh_ref[d_ppl:d_ppl + d_vh, :],
                            preferred_element_type=f32)
        w_ans_s[...] = w_dh_ref[d_ppl + d_vh:, :]

    p = ppl_ref[...].reshape(r, d_ppl)
    b_dh = b_dh_ref[...].reshape(1, h)
    bg1 = bg1_ref[...].reshape(1, h)
    bg2 = bg2_ref[...].reshape(1, h)
    bg3 = bg3_ref[...].reshape(1, h)

    row_i = jax.lax.broadcasted_iota(jnp.int32, (r, BB), 0)
    col_i = jax.lax.broadcasted_iota(jnp.int32, (r, BB), 1)
    e_mat = (row_i // n == col_i).astype(f32)
    aug = jnp.concatenate([e_mat, vh_ref[...].reshape(r, 1)], axis=1)
    ansp = jnp.dot(ans_ref[...], w_ans_s[...], preferred_element_type=f32)
    wrows = jnp.concatenate([ansp, vb_s[0:1, :]], axis=0)
    pf = (jnp.dot(p, w_dh_ref[0:d_ppl, :], preferred_element_type=f32)
          + jnp.dot(aug, wrows, preferred_element_type=f32))
    pf = jnp.maximum(pf + b_dh + vb_s[1:2, :], 0.0)

    wt = w_topo_ref[...]
    dn_t = (((1,), (1,)), ((), ()))
    pf2 = pf * pf
    wt2 = wt * wt
    invs = [1.0 / (jnp.sqrt(jax.lax.dot_general(
        pf2, wt2[k:k + 1, :], dn_t, preferred_element_type=f32)) + 1e-8)
        for k in range(kh)]

    g_all = jnp.concatenate(
        [pf * wt[k:k + 1, :] * invs[k] for k in range(kh)], axis=1)
    s_full = jax.lax.dot_general(g_all, g_all, dn_t,
                                 preferred_element_type=f32) * (1.0 / kh)

    p1 = jnp.dot(p, wg1_ref[...], preferred_element_type=f32)

    rows = jax.lax.broadcasted_iota(jnp.int32, (r, r), 0)
    cols = jax.lax.broadcasted_iota(jnp.int32, (r, r), 1)
    keep = (rows // n == cols // n) & (s_full > EPS_TOPO)
    a_hat = (jnp.where(keep, s_full, 0.0)
             + (rows == cols).astype(f32))
    deg = jnp.sum(a_hat, axis=1, keepdims=True)
    deg_t = jnp.sum(a_hat, axis=0, keepdims=True)
    l_full = a_hat * jax.lax.rsqrt(deg + 1e-8) * jax.lax.rsqrt(deg_t + 1e-8)

    h1 = jnp.maximum(
        jnp.dot(l_full, p1, preferred_element_type=f32) + bg1, 0.0)
    z2 = jnp.dot(h1, wg2_ref[...], preferred_element_type=f32)
    h2 = jnp.maximum(
        jnp.dot(l_full, z2, preferred_element_type=f32) + bg2, 0.0)
    z3 = jnp.dot(h2, wg3_ref[...], preferred_element_type=f32)
    enc = jnp.dot(l_full, z3, preferred_element_type=f32) + bg3
    out_ref[...] = enc.reshape(BB, n, h)


def kernel(ppl, visual_hint, answer, W_vh, b_vh, W_dh, b_dh, w_topo, W_g1,
           b_g1, W_g2, b_g2, W_g3, b_g3):
    B, N, D = ppl.shape
    H = W_dh.shape[1]
    VH = W_vh.shape[1]
    DA = answer.shape[1]
    DCAT = W_dh.shape[0]
    K = w_topo.shape[0]
    assert B % BB == 0 and DCAT == D + VH + DA

    grid = (B // BB,)
    const = lambda shape: pl.BlockSpec(shape, lambda i: tuple(0 for _ in shape))
    out = pl.pallas_call(
        functools.partial(_fused_kernel, n_nodes=N, k_heads=K, d_ppl=D,
                          d_vh=VH, d_ans=DA),
        grid=grid,
        in_specs=[
            pl.BlockSpec((BB, N, D), lambda i: (i, 0, 0)),
            pl.BlockSpec((BB, N, 1), lambda i: (i, 0, 0)),
            pl.BlockSpec((BB, DA), lambda i: (i, 0)),
            const((DCAT, H)), const((1, VH)), const((VH,)), const((H,)),
            const((K, H)),
            const((D, H)), const((H,)),
            const((H, H)), const((H,)),
            const((H, H)), const((H,)),
        ],
        out_specs=pl.BlockSpec((BB, N, H), lambda i: (i, 0, 0)),
        out_shape=jax.ShapeDtypeStruct((B, N, H), jnp.float32),
        scratch_shapes=[
            pltpu.VMEM((2, H), jnp.float32),
            pltpu.VMEM((DA, H), jnp.float32),
        ],
        compiler_params=pltpu.CompilerParams(
            dimension_semantics=("arbitrary",)),
    )(ppl, visual_hint, answer, W_dh, W_vh, b_vh, b_dh, w_topo,
      W_g1, b_g1, W_g2, b_g2, W_g3, b_g3)
    return out


# speedup vs baseline: 1.0718x; 1.0718x over previous
import functools

import jax
import jax.numpy as jnp
from jax.experimental import pallas as pl
from jax.experimental.pallas import tpu as pltpu

EPS_TOPO = 0.7
BB = 8


def _fused_kernel(ppl_ref, vh_ref, ans_ref, w_dh_ref, w_vh_ref, b_vh_ref,
                  b_dh_ref, w_topo_ref, wg1_ref, bg1_ref, wg2_ref,
                  bg2_ref, wg3_ref, bg3_ref, out_ref, vb_s, w_ans_s,
                  *, n_nodes, k_heads, d_ppl, d_vh, d_ans):
    f32 = jnp.float32
    n, kh = n_nodes, k_heads
    r = BB * n
    h = w_dh_ref.shape[1]

    @pl.when(pl.program_id(0) == 0)
    def _init():
        vw = jnp.concatenate(
            [w_vh_ref[...], b_vh_ref[...].reshape(1, d_vh)], axis=0)
        vb_s[...] = jnp.dot(vw, w_dh_ref[d_ppl:d_ppl + d_vh, :],
                            preferred_element_type=f32)
        w_ans_s[...] = w_dh_ref[d_ppl + d_vh:, :]

    p = ppl_ref[...].reshape(r, d_ppl)
    b_dh = b_dh_ref[...].reshape(1, h)
    bg1 = bg1_ref[...].reshape(1, h)
    bg2 = bg2_ref[...].reshape(1, h)
    bg3 = bg3_ref[...].reshape(1, h)

    row_i = jax.lax.broadcasted_iota(jnp.int32, (r, BB), 0)
    col_i = jax.lax.broadcasted_iota(jnp.int32, (r, BB), 1)
    e_mat = (row_i // n == col_i).astype(f32)
    aug = jnp.concatenate([e_mat, vh_ref[...].reshape(r, 1)], axis=1)
    ansp = jnp.dot(ans_ref[...], w_ans_s[...], preferred_element_type=f32)
    wrows = jnp.concatenate([ansp, vb_s[0:1, :]], axis=0)
    pf = (jnp.dot(p, w_dh_ref[0:d_ppl, :], preferred_element_type=f32)
          + jnp.dot(aug, wrows, preferred_element_type=f32))
    pf = jnp.maximum(pf + b_dh + vb_s[1:2, :], 0.0)

    wt = w_topo_ref[...]
    dn_t = (((1,), (1,)), ((), ()))
    pf2 = pf * pf
    wt2 = wt * wt
    def _inv_norm(ns):
        return jnp.where(ns > 0.0, jax.lax.rsqrt(ns), 0.0)

    invs = [_inv_norm(jax.lax.dot_general(
        pf2, wt2[k:k + 1, :], dn_t, preferred_element_type=f32))
        for k in range(kh)]

    g_all = jnp.concatenate(
        [pf * wt[k:k + 1, :] * invs[k] for k in range(kh)], axis=1)
    s_full = jax.lax.dot_general(g_all, g_all, dn_t,
                                 preferred_element_type=f32) * (1.0 / kh)

    p1 = jnp.dot(p, wg1_ref[...], preferred_element_type=f32)

    ls, h1s = [], []
    for b in range(BB):
        a_b = s_full[b * n:(b + 1) * n, b * n:(b + 1) * n]
        a_b = jnp.where(a_b > EPS_TOPO, a_b, 0.0)
        rows = jax.lax.broadcasted_iota(jnp.int32, (n, n), 0)
        cols = jax.lax.broadcasted_iota(jnp.int32, (n, n), 1)
        a_hat = a_b + (rows == cols).astype(f32)
        deg = jnp.sum(a_hat, axis=1, keepdims=True)
        deg_t = jnp.sum(a_hat, axis=0, keepdims=True)
        lb = a_hat * jax.lax.rsqrt(deg + 1e-8) * jax.lax.rsqrt(deg_t + 1e-8)
        ls.append(lb)
        z1 = jnp.dot(lb, p1[b * n:(b + 1) * n, :], preferred_element_type=f32)
        h1s.append(jnp.maximum(z1 + bg1, 0.0))

    h1 = jnp.concatenate(h1s, axis=0)
    z2 = jnp.dot(h1, wg2_ref[...], preferred_element_type=f32)
    h2s = [jnp.maximum(
        jnp.dot(ls[b], z2[b * n:(b + 1) * n, :], preferred_element_type=f32)
        + bg2, 0.0) for b in range(BB)]
    h2 = jnp.concatenate(h2s, axis=0)
    z3 = jnp.dot(h2, wg3_ref[...], preferred_element_type=f32)
    encs = [jnp.dot(ls[b], z3[b * n:(b + 1) * n, :], preferred_element_type=f32)
            + bg3 for b in range(BB)]
    out_ref[...] = jnp.stack(encs, axis=0)


def kernel(ppl, visual_hint, answer, W_vh, b_vh, W_dh, b_dh, w_topo, W_g1,
           b_g1, W_g2, b_g2, W_g3, b_g3):
    B, N, D = ppl.shape
    H = W_dh.shape[1]
    VH = W_vh.shape[1]
    DA = answer.shape[1]
    DCAT = W_dh.shape[0]
    K = w_topo.shape[0]
    assert B % BB == 0 and DCAT == D + VH + DA

    grid = (B // BB,)
    const = lambda shape: pl.BlockSpec(shape, lambda i: tuple(0 for _ in shape))
    out = pl.pallas_call(
        functools.partial(_fused_kernel, n_nodes=N, k_heads=K, d_ppl=D,
                          d_vh=VH, d_ans=DA),
        grid=grid,
        in_specs=[
            pl.BlockSpec((BB, N, D), lambda i: (i, 0, 0)),
            pl.BlockSpec((BB, N, 1), lambda i: (i, 0, 0)),
            pl.BlockSpec((BB, DA), lambda i: (i, 0)),
            const((DCAT, H)), const((1, VH)), const((VH,)), const((H,)),
            const((K, H)),
            const((D, H)), const((H,)),
            const((H, H)), const((H,)),
            const((H, H)), const((H,)),
        ],
        out_specs=pl.BlockSpec((BB, N, H), lambda i: (i, 0, 0)),
        out_shape=jax.ShapeDtypeStruct((B, N, H), jnp.float32),
        scratch_shapes=[
            pltpu.VMEM((2, H), jnp.float32),
            pltpu.VMEM((DA, H), jnp.float32),
        ],
        compiler_params=pltpu.CompilerParams(
            dimension_semantics=("arbitrary",)),
    )(ppl, visual_hint, answer, W_dh, W_vh, b_vh, b_dh, w_topo,
      W_g1, b_g1, W_g2, b_g2, W_g3, b_g3)
    return out


# speedup vs baseline: 1.1506x; 1.0735x over previous
import functools

import jax
import jax.numpy as jnp
from jax.experimental import pallas as pl
from jax.experimental.pallas import tpu as pltpu

EPS_TOPO = 0.7
BB = 8


def _fused_kernel(ppl_ref, vh_ref, ans_ref, w_dh_ref, w_vh_ref, b_vh_ref,
                  b_dh_ref, w_topo_ref, wg1_ref, bg1_ref, wg2_ref,
                  bg2_ref, wg3_ref, bg3_ref, out_ref, vb_s, w_ans_s,
                  *, n_nodes, k_heads, d_ppl, d_vh, d_ans):
    f32 = jnp.float32
    n, kh = n_nodes, k_heads
    r = BB * n
    h = w_dh_ref.shape[1]

    @pl.when(pl.program_id(0) == 0)
    def _init():
        vw = jnp.concatenate(
            [w_vh_ref[...], b_vh_ref[...].reshape(1, d_vh)], axis=0)
        vb_s[...] = jnp.dot(vw, w_dh_ref[d_ppl:d_ppl + d_vh, :],
                            preferred_element_type=f32)
        w_ans_s[...] = w_dh_ref[d_ppl + d_vh:, :]

    p = ppl_ref[...].reshape(r, d_ppl)
    b_dh = b_dh_ref[...].reshape(1, h)
    bg1 = bg1_ref[...].reshape(1, h)
    bg2 = bg2_ref[...].reshape(1, h)
    bg3 = bg3_ref[...].reshape(1, h)

    row_i = jax.lax.broadcasted_iota(jnp.int32, (r, BB), 0)
    col_i = jax.lax.broadcasted_iota(jnp.int32, (r, BB), 1)
    e_mat = (row_i // n == col_i).astype(f32)
    aug = jnp.concatenate([e_mat, vh_ref[...].reshape(r, 1)], axis=1)
    ansp = jnp.dot(ans_ref[...], w_ans_s[...], preferred_element_type=f32)
    wrows = jnp.concatenate([ansp, vb_s[0:1, :]], axis=0)
    pf = (jnp.dot(p, w_dh_ref[0:d_ppl, :], preferred_element_type=f32)
          + jnp.dot(aug, wrows, preferred_element_type=f32))
    pf = jnp.maximum(pf + b_dh + vb_s[1:2, :], 0.0)

    wt = w_topo_ref[...]
    dn_t = (((1,), (1,)), ((), ()))
    pf2 = pf * pf
    wt2 = wt * wt
    def _inv_norm(ns):
        return jnp.where(ns > 0.0, jax.lax.rsqrt(ns), 0.0)

    invs = [_inv_norm(jax.lax.dot_general(
        pf2, wt2[k:k + 1, :], dn_t, preferred_element_type=f32))
        for k in range(kh)]

    g_all = jnp.concatenate(
        [pf * wt[k:k + 1, :] * invs[k] for k in range(kh)], axis=1)
    n2 = 2 * n
    s_pairs = []
    for q in range(BB // 2):
        gq = g_all[q * n2:(q + 1) * n2, :]
        s_pairs.append(jax.lax.dot_general(
            gq, gq, dn_t, preferred_element_type=f32) * (1.0 / kh))

    p1 = jnp.dot(p, wg1_ref[...], preferred_element_type=f32)

    ls, h1s = [], []
    for b in range(BB):
        o = (b % 2) * n
        a_b = s_pairs[b // 2][o:o + n, o:o + n]
        a_b = jnp.where(a_b > EPS_TOPO, a_b, 0.0)
        rows = jax.lax.broadcasted_iota(jnp.int32, (n, n), 0)
        cols = jax.lax.broadcasted_iota(jnp.int32, (n, n), 1)
        a_hat = a_b + (rows == cols).astype(f32)
        deg = jnp.sum(a_hat, axis=1, keepdims=True)
        deg_t = jnp.sum(a_hat, axis=0, keepdims=True)
        lb = a_hat * jax.lax.rsqrt(deg + 1e-8) * jax.lax.rsqrt(deg_t + 1e-8)
        ls.append(lb)
        z1 = jnp.dot(lb, p1[b * n:(b + 1) * n, :], preferred_element_type=f32)
        h1s.append(jnp.maximum(z1 + bg1, 0.0))

    h1 = jnp.concatenate(h1s, axis=0)
    z2 = jnp.dot(h1, wg2_ref[...], preferred_element_type=f32)
    h2s = [jnp.maximum(
        jnp.dot(ls[b], z2[b * n:(b + 1) * n, :], preferred_element_type=f32)
        + bg2, 0.0) for b in range(BB)]
    h2 = jnp.concatenate(h2s, axis=0)
    z3 = jnp.dot(h2, wg3_ref[...], preferred_element_type=f32)
    encs = [jnp.dot(ls[b], z3[b * n:(b + 1) * n, :], preferred_element_type=f32)
            + bg3 for b in range(BB)]
    out_ref[...] = jnp.stack(encs, axis=0)


def kernel(ppl, visual_hint, answer, W_vh, b_vh, W_dh, b_dh, w_topo, W_g1,
           b_g1, W_g2, b_g2, W_g3, b_g3):
    B, N, D = ppl.shape
    H = W_dh.shape[1]
    VH = W_vh.shape[1]
    DA = answer.shape[1]
    DCAT = W_dh.shape[0]
    K = w_topo.shape[0]
    assert B % BB == 0 and DCAT == D + VH + DA

    grid = (B // BB,)
    const = lambda shape: pl.BlockSpec(shape, lambda i: tuple(0 for _ in shape))
    out = pl.pallas_call(
        functools.partial(_fused_kernel, n_nodes=N, k_heads=K, d_ppl=D,
                          d_vh=VH, d_ans=DA),
        grid=grid,
        in_specs=[
            pl.BlockSpec((BB, N, D), lambda i: (i, 0, 0)),
            pl.BlockSpec((BB, N, 1), lambda i: (i, 0, 0)),
            pl.BlockSpec((BB, DA), lambda i: (i, 0)),
            const((DCAT, H)), const((1, VH)), const((VH,)), const((H,)),
            const((K, H)),
            const((D, H)), const((H,)),
            const((H, H)), const((H,)),
            const((H, H)), const((H,)),
        ],
        out_specs=pl.BlockSpec((BB, N, H), lambda i: (i, 0, 0)),
        out_shape=jax.ShapeDtypeStruct((B, N, H), jnp.float32),
        scratch_shapes=[
            pltpu.VMEM((2, H), jnp.float32),
            pltpu.VMEM((DA, H), jnp.float32),
        ],
        compiler_params=pltpu.CompilerParams(
            dimension_semantics=("arbitrary",)),
    )(ppl, visual_hint, answer, W_dh, W_vh, b_vh, b_dh, w_topo,
      W_g1, b_g1, W_g2, b_g2, W_g3, b_g3)
    return out


# speedup vs baseline: 1.1948x; 1.0385x over previous
import functools

import jax
import jax.numpy as jnp
from jax.experimental import pallas as pl
from jax.experimental.pallas import tpu as pltpu

EPS_TOPO = 0.7
BB = 8


def _fused_kernel(ppl_ref, vh_ref, ans_ref, w_dh_ref, w_vh_ref, b_vh_ref,
                  b_dh_ref, w_topo_ref, wg1_ref, bg1_ref, wg2_ref,
                  bg2_ref, wg3_ref, bg3_ref, out_ref, vb_s, w_ans_s,
                  *, n_nodes, k_heads, d_ppl, d_vh, d_ans):
    f32 = jnp.float32
    n, kh = n_nodes, k_heads
    r = BB * n
    h = w_dh_ref.shape[1]

    @pl.when(pl.program_id(0) == 0)
    def _init():
        vw = jnp.concatenate(
            [w_vh_ref[...], b_vh_ref[...].reshape(1, d_vh)], axis=0)
        vb_s[...] = jnp.dot(vw, w_dh_ref[d_ppl:d_ppl + d_vh, :],
                            preferred_element_type=f32)
        w_ans_s[...] = w_dh_ref[d_ppl + d_vh:, :]

    p = ppl_ref[...].reshape(r, d_ppl)
    b_dh = b_dh_ref[...].reshape(1, h)
    bg1 = bg1_ref[...].reshape(1, h)
    bg2 = bg2_ref[...].reshape(1, h)
    bg3 = bg3_ref[...].reshape(1, h)

    row_i = jax.lax.broadcasted_iota(jnp.int32, (r, BB), 0)
    col_i = jax.lax.broadcasted_iota(jnp.int32, (r, BB), 1)
    e_mat = (row_i // n == col_i).astype(f32)
    aug = jnp.concatenate([e_mat, vh_ref[...].reshape(r, 1)], axis=1)
    ansp = jnp.dot(ans_ref[...], w_ans_s[...], preferred_element_type=f32)
    wrows = jnp.concatenate([ansp, vb_s[0:1, :]], axis=0)
    pf = (jnp.dot(p, w_dh_ref[0:d_ppl, :], preferred_element_type=f32)
          + jnp.dot(aug, wrows, preferred_element_type=f32))
    pf = jnp.maximum(pf + b_dh + vb_s[1:2, :], 0.0)

    wt = w_topo_ref[...]
    dn_t = (((1,), (1,)), ((), ()))
    pf2 = pf * pf
    wt2 = wt * wt
    def _inv_norm(ns):
        return jnp.where(ns > 0.0, jax.lax.rsqrt(ns), 0.0)

    invs = [_inv_norm(jax.lax.dot_general(
        pf2, wt2[k:k + 1, :], dn_t, preferred_element_type=f32))
        for k in range(kh)]
    inv_t = _inv_norm(jax.lax.dot_general(
        wt2, pf2, dn_t, preferred_element_type=f32))

    n2 = 2 * n
    s_pairs = []
    for q in range(BB // 2):
        pfq = pf[q * n2:(q + 1) * n2, :]
        acc = None
        for k in range(kh):
            c = jax.lax.dot_general(pfq * wt2[k:k + 1, :], pfq, dn_t,
                                    preferred_element_type=f32)
            c = (c * invs[k][q * n2:(q + 1) * n2, :]
                 * inv_t[k:k + 1, q * n2:(q + 1) * n2])
            acc = c if acc is None else acc + c
        s_pairs.append(acc * (1.0 / kh))

    p1 = jnp.dot(p, wg1_ref[...], preferred_element_type=f32)

    ls, h1s = [], []
    for b in range(BB):
        o = (b % 2) * n
        a_b = s_pairs[b // 2][o:o + n, o:o + n]
        a_b = jnp.where(a_b > EPS_TOPO, a_b, 0.0)
        rows = jax.lax.broadcasted_iota(jnp.int32, (n, n), 0)
        cols = jax.lax.broadcasted_iota(jnp.int32, (n, n), 1)
        a_hat = a_b + (rows == cols).astype(f32)
        deg = jnp.sum(a_hat, axis=1, keepdims=True)
        deg_t = jnp.sum(a_hat, axis=0, keepdims=True)
        lb = a_hat * jax.lax.rsqrt(deg + 1e-8) * jax.lax.rsqrt(deg_t + 1e-8)
        ls.append(lb)
        z1 = jnp.dot(lb, p1[b * n:(b + 1) * n, :], preferred_element_type=f32)
        h1s.append(jnp.maximum(z1 + bg1, 0.0))

    h1 = jnp.concatenate(h1s, axis=0)
    z2 = jnp.dot(h1, wg2_ref[...], preferred_element_type=f32)
    h2s = [jnp.maximum(
        jnp.dot(ls[b], z2[b * n:(b + 1) * n, :], preferred_element_type=f32)
        + bg2, 0.0) for b in range(BB)]
    h2 = jnp.concatenate(h2s, axis=0)
    z3 = jnp.dot(h2, wg3_ref[...], preferred_element_type=f32)
    encs = [jnp.dot(ls[b], z3[b * n:(b + 1) * n, :], preferred_element_type=f32)
            + bg3 for b in range(BB)]
    out_ref[...] = jnp.stack(encs, axis=0)


def kernel(ppl, visual_hint, answer, W_vh, b_vh, W_dh, b_dh, w_topo, W_g1,
           b_g1, W_g2, b_g2, W_g3, b_g3):
    B, N, D = ppl.shape
    H = W_dh.shape[1]
    VH = W_vh.shape[1]
    DA = answer.shape[1]
    DCAT = W_dh.shape[0]
    K = w_topo.shape[0]
    assert B % BB == 0 and DCAT == D + VH + DA

    grid = (B // BB,)
    const = lambda shape: pl.BlockSpec(shape, lambda i: tuple(0 for _ in shape))
    out = pl.pallas_call(
        functools.partial(_fused_kernel, n_nodes=N, k_heads=K, d_ppl=D,
                          d_vh=VH, d_ans=DA),
        grid=grid,
        in_specs=[
            pl.BlockSpec((BB, N, D), lambda i: (i, 0, 0)),
            pl.BlockSpec((BB, N, 1), lambda i: (i, 0, 0)),
            pl.BlockSpec((BB, DA), lambda i: (i, 0)),
            const((DCAT, H)), const((1, VH)), const((VH,)), const((H,)),
            const((K, H)),
            const((D, H)), const((H,)),
            const((H, H)), const((H,)),
            const((H, H)), const((H,)),
        ],
        out_specs=pl.BlockSpec((BB, N, H), lambda i: (i, 0, 0)),
        out_shape=jax.ShapeDtypeStruct((B, N, H), jnp.float32),
        scratch_shapes=[
            pltpu.VMEM((2, H), jnp.float32),
            pltpu.VMEM((DA, H), jnp.float32),
        ],
        compiler_params=pltpu.CompilerParams(
            dimension_semantics=("arbitrary",)),
    )(ppl, visual_hint, answer, W_dh, W_vh, b_vh, b_dh, w_topo,
      W_g1, b_g1, W_g2, b_g2, W_g3, b_g3)
    return out
